# baseline (speedup 1.0000x reference)
import functools

import jax
import jax.numpy as jnp
from jax import lax
from jax.experimental import pallas as pl
from jax.experimental.pallas import tpu as pltpu
from jax.experimental.pallas import tpu_sc as plsc

_N = 100000
_DIM = 128
_B = 16384

_NC = 2
_NS = 16
_NW = _NC * _NS
_BPW = _B // _NW
_CH = 128
_NCHUNK = _BPW // _CH


def _sc_logits_kernel(emb_hbm, u_hbm, v_hbm, w_hbm, out_hbm,
                      u_c, v_c, ru, rv, w_v, res_v, sem):
    wid = lax.axis_index("s") * _NC + lax.axis_index("c")
    base = wid * _BPW

    pltpu.sync_copy(w_hbm.at[pl.ds(base, _BPW)], w_v)

    row_base = jnp.arange(16, dtype=jnp.int32)

    for c in range(_NCHUNK):
        off = base + c * _CH
        pltpu.sync_copy(u_hbm.at[pl.ds(off, _CH)], u_c)
        pltpu.sync_copy(v_hbm.at[pl.ds(off, _CH)], v_c)
        cp_u = pltpu.async_copy(emb_hbm.at[u_c], ru, sem)
        cp_v = pltpu.async_copy(emb_hbm.at[v_c], rv, sem)
        cp_u.wait()
        cp_v.wait()

        for g in range(_CH // 16):
            rows = row_base + (g * 16)

            def body(d, acc):
                cols = jnp.full((16,), d, dtype=jnp.int32)
                a = plsc.load_gather(ru, [rows, cols])
                b = plsc.load_gather(rv, [rows, cols])
                return acc + a * b

            acc = lax.fori_loop(0, _DIM, body, jnp.zeros((16,), jnp.float32))
            sl = pl.ds(c * _CH + g * 16, 16)
            res_v[sl] = acc * w_v[sl]

    pltpu.sync_copy(res_v, out_hbm.at[pl.ds(base, _BPW)])


def _sc_logits(u, v, w, emb1):
    mesh = plsc.VectorSubcoreMesh(core_axis_name="c", subcore_axis_name="s")
    return pl.kernel(
        _sc_logits_kernel,
        out_type=jax.ShapeDtypeStruct((_B,), jnp.float32),
        mesh=mesh,
        scratch_types=[
            pltpu.VMEM((_CH,), jnp.int32),
            pltpu.VMEM((_CH,), jnp.int32),
            pltpu.VMEM((_CH, _DIM), jnp.float32),
            pltpu.VMEM((_CH, _DIM), jnp.float32),
            pltpu.VMEM((_BPW,), jnp.float32),
            pltpu.VMEM((_BPW,), jnp.float32),
            pltpu.SemaphoreType.DMA,
        ],
        compiler_params=pltpu.CompilerParams(needs_layout_passes=False),
    )(emb1, u, v, w)


def _tc_loss_body(x_ref, o_ref):
    x = x_ref[...]
    ls = jnp.minimum(x, 0.0) - jnp.log1p(jnp.exp(-jnp.abs(x)))
    o_ref[0, 0] = -jnp.sum(ls) * (1.0 / _B)


def _tc_loss(x):
    x2 = x.reshape(_B // _DIM, _DIM)
    out = pl.pallas_call(
        _tc_loss_body,
        out_shape=jax.ShapeDtypeStruct((1, 1), jnp.float32),
        out_specs=pl.BlockSpec(memory_space=pltpu.SMEM),
    )(x2)
    return out[0, 0]


@jax.jit
def kernel(u, v, w, emb1):
    x = _sc_logits(u, v, w, emb1)
    return _tc_loss(x)


# speedup vs baseline: 2.6638x; 2.6638x over previous
import functools

import jax
import jax.numpy as jnp
from jax import lax
from jax.experimental import pallas as pl
from jax.experimental.pallas import tpu as pltpu
from jax.experimental.pallas import tpu_sc as plsc

_N = 100000
_DIM = 128
_B = 16384

_NC = 2
_NS = 16
_NW = _NC * _NS
_BPW = _B // _NW
_CH = 128
_NCHUNK = _BPW // _CH


def _sc_logits_kernel(emb_hbm, u_hbm, v_hbm, w_hbm, out_hbm,
                      u_v, v_v, ru0, rv0, ru1, rv1, w_v, res_v, sem0, sem1):
    wid = lax.axis_index("s") * _NC + lax.axis_index("c")
    base = wid * _BPW

    pltpu.sync_copy(u_hbm.at[pl.ds(base, _BPW)], u_v)
    pltpu.sync_copy(v_hbm.at[pl.ds(base, _BPW)], v_v)
    pltpu.sync_copy(w_hbm.at[pl.ds(base, _BPW)], w_v)

    bufs = ((ru0, rv0, sem0), (ru1, rv1, sem1))

    def start_chunk(c):
        ru, rv, sem = bufs[c % 2]
        sl = pl.ds(c * _CH, _CH)
        cu = pltpu.async_copy(emb_hbm.at[u_v.at[sl]], ru, sem)
        cv = pltpu.async_copy(emb_hbm.at[v_v.at[sl]], rv, sem)
        return cu, cv

    lane = jnp.arange(16, dtype=jnp.int32)
    pend = start_chunk(0)

    for c in range(_NCHUNK):
        ru, rv, _ = bufs[c % 2]
        pend[0].wait()
        pend[1].wait()
        if c + 1 < _NCHUNK:
            pend = start_chunk(c + 1)

        for g in range(_CH // 16):
            rows = lane + (g * 16)
            zero = jnp.zeros((16,), jnp.float32)

            @plsc.parallel_loop(0, _DIM, step=2, unroll=4, carry=(zero, zero))
            def acc_loop(d, acc):
                a0, a1 = acc
                c0 = (lane + d) & (_DIM - 1)
                c1 = (lane + d + 1) & (_DIM - 1)
                a0 = a0 + plsc.load_gather(ru, [rows, c0]) * plsc.load_gather(rv, [rows, c0])
                a1 = a1 + plsc.load_gather(ru, [rows, c1]) * plsc.load_gather(rv, [rows, c1])
                return a0, a1

            sl = pl.ds(c * _CH + g * 16, 16)
            res_v[sl] = (acc_loop[0] + acc_loop[1]) * w_v[sl]

    pltpu.sync_copy(res_v, out_hbm.at[pl.ds(base, _BPW)])


def _sc_logits(u, v, w, emb1):
    mesh = plsc.VectorSubcoreMesh(core_axis_name="c", subcore_axis_name="s")
    return pl.kernel(
        _sc_logits_kernel,
        out_type=jax.ShapeDtypeStruct((_B,), jnp.float32),
        mesh=mesh,
        scratch_types=[
            pltpu.VMEM((_BPW,), jnp.int32),
            pltpu.VMEM((_BPW,), jnp.int32),
            pltpu.VMEM((_CH, _DIM), jnp.float32),
            pltpu.VMEM((_CH, _DIM), jnp.float32),
            pltpu.VMEM((_CH, _DIM), jnp.float32),
            pltpu.VMEM((_CH, _DIM), jnp.float32),
            pltpu.VMEM((_BPW,), jnp.float32),
            pltpu.VMEM((_BPW,), jnp.float32),
            pltpu.SemaphoreType.DMA,
            pltpu.SemaphoreType.DMA,
        ],
        compiler_params=pltpu.CompilerParams(needs_layout_passes=False),
    )(emb1, u, v, w)


def _tc_loss_body(x_ref, o_ref):
    x = x_ref[...]
    ls = jnp.minimum(x, 0.0) - jnp.log1p(jnp.exp(-jnp.abs(x)))
    o_ref[0, 0] = -jnp.sum(ls) * (1.0 / _B)


def _tc_loss(x):
    x2 = x.reshape(_B // _DIM, _DIM)
    out = pl.pallas_call(
        _tc_loss_body,
        out_shape=jax.ShapeDtypeStruct((1, 1), jnp.float32),
        out_specs=pl.BlockSpec(memory_space=pltpu.SMEM),
    )(x2)
    return out[0, 0]


@jax.jit
def kernel(u, v, w, emb1):
    x = _sc_logits(u, v, w, emb1)
    return _tc_loss(x)


# speedup vs baseline: 2.6728x; 1.0034x over previous
import functools

import jax
import jax.numpy as jnp
from jax import lax
from jax.experimental import pallas as pl
from jax.experimental.pallas import tpu as pltpu
from jax.experimental.pallas import tpu_sc as plsc

_N = 100000
_DIM = 128
_B = 16384

_NC = 2
_NS = 16
_NW = _NC * _NS
_BPW = _B // _NW
_CH = 128
_NCHUNK = _BPW // _CH


def _sc_logits_kernel(emb_hbm, u_hbm, v_hbm, w_hbm, out_hbm,
                      u_v, v_v, ru0, rv0, ru1, rv1, w_v, res_v, sem0, sem1):
    wid = lax.axis_index("s") * _NC + lax.axis_index("c")
    base = wid * _BPW

    pltpu.sync_copy(u_hbm.at[pl.ds(base, _BPW)], u_v)
    pltpu.sync_copy(v_hbm.at[pl.ds(base, _BPW)], v_v)
    pltpu.sync_copy(w_hbm.at[pl.ds(base, _BPW)], w_v)

    bufs = ((ru0, rv0, sem0), (ru1, rv1, sem1))

    def start_chunk(c):
        ru, rv, sem = bufs[c % 2]
        sl = pl.ds(c * _CH, _CH)
        cu = pltpu.async_copy(emb_hbm.at[u_v.at[sl]], ru, sem)
        cv = pltpu.async_copy(emb_hbm.at[v_v.at[sl]], rv, sem)
        return cu, cv

    lane = jnp.arange(16, dtype=jnp.int32)
    pend = start_chunk(0)

    for c in range(_NCHUNK):
        ru, rv, _ = bufs[c % 2]
        pend[0].wait()
        pend[1].wait()
        if c + 1 < _NCHUNK:
            pend = start_chunk(c + 1)

        for g in range(_CH // 16):
            rows = lane + (g * 16)
            zero = jnp.zeros((16,), jnp.float32)

            @plsc.parallel_loop(0, _DIM, step=2, unroll=4, carry=(zero, zero))
            def acc_loop(d, acc):
                a0, a1 = acc
                c0 = (lane + d) & (_DIM - 1)
                c1 = (lane + d + 1) & (_DIM - 1)
                a0 = a0 + plsc.load_gather(ru, [rows, c0]) * plsc.load_gather(rv, [rows, c0])
                a1 = a1 + plsc.load_gather(ru, [rows, c1]) * plsc.load_gather(rv, [rows, c1])
                return a0, a1

            sl = pl.ds(c * _CH + g * 16, 16)
            res_v[sl] = (acc_loop[0] + acc_loop[1]) * w_v[sl]

    pltpu.sync_copy(res_v, out_hbm.at[pl.ds(base, _BPW)])


def _sc_logits(u, v, w, emb1):
    mesh = plsc.VectorSubcoreMesh(core_axis_name="c", subcore_axis_name="s")
    return pl.kernel(
        _sc_logits_kernel,
        out_type=jax.ShapeDtypeStruct((_B,), jnp.float32),
        mesh=mesh,
        scratch_types=[
            pltpu.VMEM((_BPW,), jnp.int32),
            pltpu.VMEM((_BPW,), jnp.int32),
            pltpu.VMEM((_CH, _DIM), jnp.float32),
            pltpu.VMEM((_CH, _DIM), jnp.float32),
            pltpu.VMEM((_CH, _DIM), jnp.float32),
            pltpu.VMEM((_CH, _DIM), jnp.float32),
            pltpu.VMEM((_BPW,), jnp.float32),
            pltpu.VMEM((_BPW,), jnp.float32),
            pltpu.SemaphoreType.DMA,
            pltpu.SemaphoreType.DMA,
        ],
        compiler_params=pltpu.CompilerParams(
            needs_layout_passes=False, skip_device_barrier=True
        ),
    )(emb1, u, v, w)


def _tc_loss_body(x_ref, o_ref):
    x = x_ref[...]
    ls = jnp.minimum(x, 0.0) - jnp.log1p(jnp.exp(-jnp.abs(x)))
    o_ref[0, 0] = -jnp.sum(ls) * (1.0 / _B)


def _tc_loss(x):
    x2 = x.reshape(_B // _DIM, _DIM)
    out = pl.pallas_call(
        _tc_loss_body,
        out_shape=jax.ShapeDtypeStruct((1, 1), jnp.float32),
        out_specs=pl.BlockSpec(memory_space=pltpu.SMEM),
        compiler_params=pltpu.CompilerParams(skip_device_barrier=True),
    )(x2)
    return out[0, 0]


@jax.jit
def kernel(u, v, w, emb1):
    x = _sc_logits(u, v, w, emb1)
    return _tc_loss(x)
